# Initial kernel scaffold; baseline (speedup 1.0000x reference)
import functools

import jax
import jax.numpy as jnp
from jax import lax
from jax.experimental import pallas as pl
from jax.experimental.pallas import tpu as pltpu
from jax.experimental.pallas import tpu_sc as plsc

B, H, W, C = 4, 112, 112, 96
W_OUT = 2 * W
ROW = W * C
OROW = 2 * W_OUT * C
N_ROWS = B * H
LANES = 16
NC, NS = 2, 16
NW = NC * NS
ROWS_PER_W = N_ROWS // NW


def _unpool_body(in_hbm, mask_hbm, out_hbm, valv, idxv, outb):
    wid = lax.axis_index("s") * NC + lax.axis_index("c")

    def do_row(j, _):
        r = wid * ROWS_PER_W + j
        h = lax.rem(r, H)
        off = h * OROW

        pltpu.sync_copy(in_hbm.at[r], valv)
        pltpu.sync_copy(mask_hbm.at[r], idxv)

        zeros = jnp.zeros((LANES,), jnp.float32)

        def zero_chunk(i, _):
            base = i * (4 * LANES)
            for k in range(4):
                outb[pl.ds(base + k * LANES, LANES)] = zeros
            return 0

        lax.fori_loop(0, OROW // (4 * LANES), zero_chunk, 0)

        def scat_chunk(i, _):
            base = i * (2 * LANES)
            for k in range(2):
                v = valv[pl.ds(base + k * LANES, LANES)]
                ix = idxv[pl.ds(base + k * LANES, LANES)] - off
                plsc.store_scatter(outb, [ix], v)
            return 0

        lax.fori_loop(0, ROW // (2 * LANES), scat_chunk, 0)

        pltpu.sync_copy(outb, out_hbm.at[r])
        return 0

    lax.fori_loop(0, ROWS_PER_W, do_row, 0)


@jax.jit
def _unpool(in_flat, mask_flat):
    mesh = plsc.VectorSubcoreMesh(core_axis_name="c", subcore_axis_name="s")
    return pl.kernel(
        _unpool_body,
        out_type=jax.ShapeDtypeStruct((N_ROWS, OROW), jnp.float32),
        mesh=mesh,
        scratch_types=[
            pltpu.VMEM((ROW,), jnp.float32),
            pltpu.VMEM((ROW,), jnp.int32),
            pltpu.VMEM((OROW,), jnp.float32),
        ],
    )(in_flat, mask_flat)


def kernel(inputs, mask):
    in_flat = inputs.reshape(N_ROWS, ROW)
    mask_flat = mask.astype(jnp.int32).reshape(N_ROWS, ROW)
    out = _unpool(in_flat, mask_flat)
    return out.reshape(B, 2 * H, W_OUT, C)


# speedup vs baseline: 36.1244x; 36.1244x over previous
import functools

import jax
import jax.numpy as jnp
from jax import lax
from jax.experimental import pallas as pl
from jax.experimental.pallas import tpu as pltpu
from jax.experimental.pallas import tpu_sc as plsc

B, H, W, C = 4, 112, 112, 96
W_OUT = 2 * W
ROW = W * C
OROW = 2 * W_OUT * C
N_ROWS = B * H
LANES = 16
NC, NS = 2, 16
NW = NC * NS
ROWS_PER_W = N_ROWS // NW


def _unpool_body(in_hbm, mask_hbm, out_hbm, valv, idxv, outb):
    wid = lax.axis_index("s") * NC + lax.axis_index("c")

    def do_row(j, _):
        r = wid * ROWS_PER_W + j
        h = lax.rem(r, H)
        off = h * OROW

        pltpu.sync_copy(in_hbm.at[r], valv)
        pltpu.sync_copy(mask_hbm.at[r], idxv)

        zeros = jnp.zeros((LANES,), jnp.float32)

        def zero_chunk(i, _):
            base = i * (4 * LANES)
            for k in range(4):
                outb[pl.ds(base + k * LANES, LANES)] = zeros
            return 0

        lax.fori_loop(0, OROW // (4 * LANES), zero_chunk, 0)

        def scat_chunk(i, _):
            base = i * (2 * LANES)
            for k in range(2):
                v = valv[pl.ds(base + k * LANES, LANES)]
                ix = idxv[pl.ds(base + k * LANES, LANES)] - off
                plsc.store_scatter(outb, [ix], v)
            return 0

        lax.fori_loop(0, ROW // (2 * LANES), scat_chunk, 0)

        pltpu.sync_copy(outb, out_hbm.at[r])
        return 0

    lax.fori_loop(0, ROWS_PER_W, do_row, 0)


@jax.jit
def _unpool(in_flat, mask_flat):
    mesh = plsc.VectorSubcoreMesh(core_axis_name="c", subcore_axis_name="s")
    return pl.kernel(
        _unpool_body,
        out_type=jax.ShapeDtypeStruct((N_ROWS, OROW), jnp.float32),
        mesh=mesh,
        scratch_types=[
            pltpu.VMEM((ROW,), jnp.float32),
            pltpu.VMEM((ROW,), jnp.int32),
            pltpu.VMEM((OROW,), jnp.float32),
        ],
        compiler_params=pltpu.CompilerParams(needs_layout_passes=False),
    )(in_flat, mask_flat)


def kernel(inputs, mask):
    in_flat = inputs.reshape(N_ROWS, ROW)
    mask_flat = mask.astype(jnp.int32).reshape(N_ROWS, ROW)
    out = _unpool(in_flat, mask_flat)
    return out.reshape(B, 2 * H, W_OUT, C)


# speedup vs baseline: 46.8517x; 1.2970x over previous
import jax
import jax.numpy as jnp
from jax import lax
from jax.experimental import pallas as pl
from jax.experimental.pallas import tpu as pltpu
from jax.experimental.pallas import tpu_sc as plsc

B, H, W, C = 4, 112, 112, 96
W_OUT = 2 * W
ROW = W * C
HALF = W_OUT * C
OROW = 2 * HALF
N_ROWS = B * H
L = 16
NC, NS = 2, 16
NW = NC * NS
RPW = N_ROWS // NW
KCH = C // L


def _unpool_body(in_hbm, mask_hbm, out_hbm,
                 val0, val1, idx0, idx1, out0, out1,
                 sin0, sin1, sout0, sout1):
    wid = lax.axis_index("s") * NC + lax.axis_index("c")
    r0 = wid * RPW

    vals = (val0, val1)
    idxs = (idx0, idx1)
    outs = (out0, out1)
    sins = (sin0, sin1)
    souts = (sout0, sout1)

    lane = lax.iota(jnp.int32, L)

    def in_copies(j):
        p = j % 2
        r = r0 + j
        return (pltpu.make_async_copy(in_hbm.at[r], vals[p], sins[p]),
                pltpu.make_async_copy(mask_hbm.at[r], idxs[p], sins[p]))

    def out_copy(j):
        p = j % 2
        return pltpu.make_async_copy(outs[p], out_hbm.at[r0 + j], souts[p])

    for cp in in_copies(0):
        cp.start()

    for j in range(RPW):
        p = j % 2
        r = r0 + j
        h = lax.rem(r, H)
        off = h * OROW

        for cp in in_copies(j):
            cp.wait()
        if j + 1 < RPW:
            for cp in in_copies(j + 1):
                cp.start()
        if j >= 2:
            out_copy(j - 2).wait()

        valv, idxv = vals[p], idxs[p]
        outb = outs[p]

        def do_w(w, _):
            base_in = w * C
            for k in range(KCH):
                v = valv[pl.ds(base_in + k * L, L)]
                ixl = idxv[pl.ds(base_in + k * L, L)] - off
                for dh in range(2):
                    for dw in range(2):
                        pos = dh * HALF + (2 * w + dw) * C + k * L
                        sel = ixl == (pos + lane)
                        outb[pl.ds(pos, L)] = jnp.where(sel, v, 0.0)
            return 0

        lax.fori_loop(0, W, do_w, 0)

        out_copy(j).start()

    out_copy(RPW - 2).wait()
    out_copy(RPW - 1).wait()


@jax.jit
def _unpool(in_flat, mask_flat):
    mesh = plsc.VectorSubcoreMesh(core_axis_name="c", subcore_axis_name="s")
    return pl.kernel(
        _unpool_body,
        out_type=jax.ShapeDtypeStruct((N_ROWS, OROW), jnp.float32),
        mesh=mesh,
        scratch_types=[
            pltpu.VMEM((ROW,), jnp.float32),
            pltpu.VMEM((ROW,), jnp.float32),
            pltpu.VMEM((ROW,), jnp.int32),
            pltpu.VMEM((ROW,), jnp.int32),
            pltpu.VMEM((OROW,), jnp.float32),
            pltpu.VMEM((OROW,), jnp.float32),
            pltpu.SemaphoreType.DMA,
            pltpu.SemaphoreType.DMA,
            pltpu.SemaphoreType.DMA,
            pltpu.SemaphoreType.DMA,
        ],
        compiler_params=pltpu.CompilerParams(needs_layout_passes=False),
    )(in_flat, mask_flat)


def kernel(inputs, mask):
    in_flat = inputs.reshape(N_ROWS, ROW)
    mask_flat = mask.astype(jnp.int32).reshape(N_ROWS, ROW)
    out = _unpool(in_flat, mask_flat)
    return out.reshape(B, 2 * H, W_OUT, C)


# speedup vs baseline: 50.7318x; 1.0828x over previous
import jax
import jax.numpy as jnp
from jax import lax
from jax.experimental import pallas as pl
from jax.experimental.pallas import tpu as pltpu
from jax.experimental.pallas import tpu_sc as plsc

B, H, W, C = 4, 112, 112, 96
W_OUT = 2 * W
ROW = W * C
HALF = W_OUT * C
OROW = 2 * HALF
N_ROWS = B * H
L = 16
NC, NS = 2, 16
NW = NC * NS
RPW = N_ROWS // NW
KCH = C // L


def _unpool_body(in_hbm, mask_hbm, out_hbm,
                 val0, val1, idx0, idx1, out0, out1,
                 sin0, sin1, sout0, sout1):
    wid = lax.axis_index("s") * NC + lax.axis_index("c")
    r0 = wid * RPW

    vals = (val0, val1)
    idxs = (idx0, idx1)
    outs = (out0, out1)
    sins = (sin0, sin1)
    souts = (sout0, sout1)

    lane = lax.iota(jnp.int32, L)

    def in_copies(j):
        p = j % 2
        r = r0 + j
        return (pltpu.make_async_copy(in_hbm.at[r], vals[p], sins[p]),
                pltpu.make_async_copy(mask_hbm.at[r], idxs[p], sins[p]))

    def out_copy(j):
        p = j % 2
        return pltpu.make_async_copy(outs[p], out_hbm.at[r0 + j], souts[p])

    for cp in in_copies(0):
        cp.start()

    for j in range(RPW):
        p = j % 2
        r = r0 + j
        h = lax.rem(r, H)
        off = h * OROW

        for cp in in_copies(j):
            cp.wait()
        if j + 1 < RPW:
            for cp in in_copies(j + 1):
                cp.start()
        if j >= 2:
            out_copy(j - 2).wait()

        valv, idxv = vals[p], idxs[p]
        outb = outs[p]
        offv = off + lane

        @plsc.parallel_loop(0, W, step=1, unroll=2)
        def do_w(w):
            base_in = w * C
            for k in range(KCH):
                v = valv[pl.ds(base_in + k * L, L)]
                t = idxv[pl.ds(base_in + k * L, L)] - offv
                for dh in range(2):
                    for dw in range(2):
                        pos = dh * HALF + (2 * w + dw) * C + k * L
                        outb[pl.ds(pos, L)] = jnp.where(t == pos, v, 0.0)

        out_copy(j).start()

    out_copy(RPW - 2).wait()
    out_copy(RPW - 1).wait()


@jax.jit
def _unpool(in_flat, mask_flat):
    mesh = plsc.VectorSubcoreMesh(core_axis_name="c", subcore_axis_name="s")
    return pl.kernel(
        _unpool_body,
        out_type=jax.ShapeDtypeStruct((N_ROWS, OROW), jnp.float32),
        mesh=mesh,
        scratch_types=[
            pltpu.VMEM((ROW,), jnp.float32),
            pltpu.VMEM((ROW,), jnp.float32),
            pltpu.VMEM((ROW,), jnp.int32),
            pltpu.VMEM((ROW,), jnp.int32),
            pltpu.VMEM((OROW,), jnp.float32),
            pltpu.VMEM((OROW,), jnp.float32),
            pltpu.SemaphoreType.DMA,
            pltpu.SemaphoreType.DMA,
            pltpu.SemaphoreType.DMA,
            pltpu.SemaphoreType.DMA,
        ],
        compiler_params=pltpu.CompilerParams(needs_layout_passes=False),
    )(in_flat, mask_flat)


def kernel(inputs, mask):
    in_flat = inputs.reshape(N_ROWS, ROW)
    mask_flat = mask.astype(jnp.int32).reshape(N_ROWS, ROW)
    out = _unpool(in_flat, mask_flat)
    return out.reshape(B, 2 * H, W_OUT, C)


# speedup vs baseline: 212.9326x; 4.1972x over previous
import jax
import jax.numpy as jnp
from jax import lax
from jax.experimental import pallas as pl
from jax.experimental.pallas import tpu as pltpu
from jax.experimental.pallas import tpu_sc as plsc

B, H, W, C = 4, 112, 112, 96
W_OUT = 2 * W
HALF = W_OUT * C
N_ROWS = B * H
L = 16
NC, NS = 2, 16
NW = NC * NS
RPW = N_ROWS // NW
NCH = W // L
NOB = 3


def _unpool_body(in_hbm, mask_hbm, out_hbm,
                 val0, val1, mk0, mk1, ob0, ob1, ob2,
                 sin0, sin1, so0, so1, so2):
    wid = lax.axis_index("s") * NC + lax.axis_index("c")
    r0 = wid * RPW

    vals = (val0, val1)
    mks = (mk0, mk1)
    obs = (ob0, ob1, ob2)
    sins = (sin0, sin1)
    sos = (so0, so1, so2)

    iota = lax.iota(jnp.int32, L)
    e192 = 192 * iota
    iow = [2 * iota + (2 * w0 + dw) for w0 in range(0, W, L) for dw in range(2)]

    def in_copies(j):
        p = j % 2
        r = r0 + j
        b, h = r // H, lax.rem(r, H)
        return (pltpu.make_async_copy(in_hbm.at[b, h], vals[p], sins[p]),
                pltpu.make_async_copy(mask_hbm.at[b, h], mks[p], sins[p]))

    def out_copy(s):
        j, dh = s // 2, s % 2
        r = r0 + j
        b, h = r // H, lax.rem(r, H)
        return pltpu.make_async_copy(obs[s % NOB], out_hbm.at[b, 2 * h + dh],
                                     sos[s % NOB])

    for cp in in_copies(0):
        cp.start()

    for j in range(RPW):
        p = j % 2
        r = r0 + j
        h = lax.rem(r, H)

        for cp in in_copies(j):
            cp.wait()
        if j + 1 < RPW:
            for cp in in_copies(j + 1):
                cp.start()

        valb, mkb = vals[p], mks[p]

        for dh in range(2):
            s = 2 * j + dh
            if s >= NOB:
                out_copy(s - NOB).wait()
            ob = obs[s % NOB]
            base = (2 * h + dh) * HALF

            @plsc.parallel_loop(0, C, step=1, unroll=1)
            def do_c(c):
                cvec = jnp.full((L,), c, jnp.int32)
                bc = base + c
                for k in range(NCH):
                    w0 = k * L
                    v = valb[c, pl.ds(w0, L)]
                    t = mkb[c, pl.ds(w0, L)] - e192
                    for dw in range(2):
                        x = jnp.where(t == bc + 192 * w0 + 96 * dw, v, 0.0)
                        plsc.store_scatter(ob, [cvec, iow[2 * k + dw]], x)

            out_copy(s).start()

    for s in range(2 * RPW - NOB, 2 * RPW):
        out_copy(s).wait()


@jax.jit
def _unpool(in_t, mask_t):
    mesh = plsc.VectorSubcoreMesh(core_axis_name="c", subcore_axis_name="s")
    return pl.kernel(
        _unpool_body,
        out_type=jax.ShapeDtypeStruct((B, 2 * H, C, W_OUT), jnp.float32),
        mesh=mesh,
        scratch_types=[
            pltpu.VMEM((C, W), jnp.float32),
            pltpu.VMEM((C, W), jnp.float32),
            pltpu.VMEM((C, W), jnp.int32),
            pltpu.VMEM((C, W), jnp.int32),
            pltpu.VMEM((C, W_OUT), jnp.float32),
            pltpu.VMEM((C, W_OUT), jnp.float32),
            pltpu.VMEM((C, W_OUT), jnp.float32),
            pltpu.SemaphoreType.DMA,
            pltpu.SemaphoreType.DMA,
            pltpu.SemaphoreType.DMA,
            pltpu.SemaphoreType.DMA,
            pltpu.SemaphoreType.DMA,
        ],
        compiler_params=pltpu.CompilerParams(
            needs_layout_passes=False, use_tc_tiling_on_sc=True),
    )(in_t, mask_t)


def kernel(inputs, mask):
    in_t = jnp.transpose(inputs, (0, 1, 3, 2))
    mask_t = jnp.transpose(mask.astype(jnp.int32), (0, 1, 3, 2))
    out_t = _unpool(in_t, mask_t)
    return jnp.transpose(out_t, (0, 1, 3, 2))
